# baseline (speedup 1.0000x reference)
import functools

import jax
import jax.numpy as jnp
from jax import lax
from jax.experimental import pallas as pl
from jax.experimental.pallas import tpu as pltpu
from jax.experimental.pallas import tpu_sc as plsc

B, S = 64, 2048
F0_DIM, PH_DIM, MIDI_DIM = 64, 192, 64
NUM_PHONES, NUM_MIDI = 100, 128
D_TOT = F0_DIM + PH_DIM + MIDI_DIM
PH_STRIDE, MIDI_STRIDE = PH_DIM + 1, MIDI_DIM + 1

CHUNK = 128
SG = CHUNK // 16
N_CHUNK = S // CHUNK


def _sc_body(f0_h, plab_h, mlab_h, ptab_h, mtab_h, w_h, bias_h, out_h,
             ptab_v, mtab_v, w_v, bias_v, labp_v, labm_v, f0_v,
             out0_v, out1_v, sem0, sem1):
    nc = 2
    wid = lax.axis_index("s") * nc + lax.axis_index("c")

    pltpu.sync_copy(ptab_h, ptab_v)
    pltpu.sync_copy(mtab_h, mtab_v)
    pltpu.sync_copy(w_h, w_v)
    pltpu.sync_copy(bias_h, bias_v)

    def compute(buf, off):
        fvs = [f0_v[pl.ds(off + sg * 16, 16)] for sg in range(SG)]

        @plsc.parallel_loop(0, F0_DIM // 16, unroll=1)
        def _(g):
            wvec = w_v[pl.ds(g * 16, 16)]
            bvec = bias_v[pl.ds(g * 16, 16)]
            for j in range(16):
                w = wvec[j]
                bb = bvec[j]
                for sg in range(SG):
                    buf[g * 16 + j, pl.ds(sg * 16, 16)] = fvs[sg] * w + bb

        basep = tuple(
            labp_v[pl.ds(off + sg * 16, 16)] * PH_STRIDE for sg in range(SG))

        @plsc.parallel_loop(0, PH_DIM, unroll=12, carry=basep)
        def _(d, idxs):
            for sg in range(SG):
                buf[F0_DIM + d, pl.ds(sg * 16, 16)] = plsc.load_gather(
                    ptab_v, [idxs[sg]])
            return tuple(ix + 1 for ix in idxs)

        basem = tuple(
            labm_v[pl.ds(off + sg * 16, 16)] * MIDI_STRIDE for sg in range(SG))

        @plsc.parallel_loop(0, MIDI_DIM, unroll=8, carry=basem)
        def _(d, idxs):
            for sg in range(SG):
                buf[F0_DIM + PH_DIM + d, pl.ds(sg * 16, 16)] = \
                    plsc.load_gather(mtab_v, [idxs[sg]])
            return tuple(ix + 1 for ix in idxs)

    n_iter = 2 * (N_CHUNK // 2)

    def body(i, _):
        b = wid * 2 + i // (N_CHUNK // 2)
        ic = i % (N_CHUNK // 2)

        @pl.when(ic == 0)
        def _():
            pltpu.sync_copy(plab_h.at[b], labp_v)
            pltpu.sync_copy(mlab_h.at[b], labm_v)
            pltpu.sync_copy(f0_h.at[b], f0_v)

        off0 = ic * (2 * CHUNK)

        @pl.when(i > 0)
        def _():
            pltpu.make_async_copy(
                out0_v, out_h.at[b, :, pl.ds(off0, CHUNK)], sem0).wait()

        compute(out0_v, off0)
        pltpu.async_copy(out0_v, out_h.at[b, :, pl.ds(off0, CHUNK)], sem0)

        @pl.when(i > 0)
        def _():
            pltpu.make_async_copy(
                out1_v, out_h.at[b, :, pl.ds(off0 + CHUNK, CHUNK)],
                sem1).wait()

        compute(out1_v, off0 + CHUNK)
        pltpu.async_copy(
            out1_v, out_h.at[b, :, pl.ds(off0 + CHUNK, CHUNK)], sem1)
        return 0

    lax.fori_loop(0, n_iter, body, 0)

    pltpu.make_async_copy(
        out0_v, out_h.at[0, :, pl.ds(0, CHUNK)], sem0).wait()
    pltpu.make_async_copy(
        out1_v, out_h.at[0, :, pl.ds(0, CHUNK)], sem1).wait()


@jax.jit
def _run(f0_2d, plab, mlab, ptab_flat, mtab_flat, w_flat, bias):
    mesh = plsc.VectorSubcoreMesh(core_axis_name="c", subcore_axis_name="s")
    sck = functools.partial(
        pl.kernel,
        mesh=mesh,
        compiler_params=pltpu.CompilerParams(needs_layout_passes=False),
        out_type=jax.ShapeDtypeStruct((B, D_TOT, S), jnp.float32),
        scratch_types=[
            pltpu.VMEM((NUM_PHONES * PH_STRIDE,), jnp.float32),
            pltpu.VMEM((NUM_MIDI * MIDI_STRIDE,), jnp.float32),
            pltpu.VMEM((F0_DIM,), jnp.float32),
            pltpu.VMEM((F0_DIM,), jnp.float32),
            pltpu.VMEM((S,), jnp.int32),
            pltpu.VMEM((S,), jnp.int32),
            pltpu.VMEM((S,), jnp.float32),
            pltpu.VMEM((D_TOT, CHUNK), jnp.float32),
            pltpu.VMEM((D_TOT, CHUNK), jnp.float32),
            pltpu.SemaphoreType.DMA,
            pltpu.SemaphoreType.DMA,
        ],
    )(_sc_body)
    return sck(f0_2d, plab, mlab, ptab_flat, mtab_flat, w_flat, bias)


def kernel(f0, phone_label, phone_duration, midi_label, W_f0, b_f0,
           phone_table, midi_table):
    del phone_duration
    f0_2d = f0.reshape(B, S)
    plab = phone_label.astype(jnp.int32)
    mlab = midi_label.astype(jnp.int32)
    ptab_pad = jnp.pad(phone_table, ((0, 0), (0, 1))).reshape(-1)
    mtab_pad = jnp.pad(midi_table, ((0, 0), (0, 1))).reshape(-1)
    return _run(f0_2d, plab, mlab, ptab_pad, mtab_pad,
                W_f0.reshape(-1), b_f0)


# speedup vs baseline: 1.1456x; 1.1456x over previous
import functools

import jax
import jax.numpy as jnp
from jax import lax
from jax.experimental import pallas as pl
from jax.experimental.pallas import tpu as pltpu
from jax.experimental.pallas import tpu_sc as plsc

B, S = 64, 2048
F0_DIM, PH_DIM, MIDI_DIM = 64, 192, 64
NUM_PHONES, NUM_MIDI = 100, 128
D_TOT = F0_DIM + PH_DIM + MIDI_DIM
PH_STRIDE, MIDI_STRIDE = PH_DIM + 1, MIDI_DIM + 1

CHUNK = 128
SG = CHUNK // 16
N_CHUNK = S // CHUNK


def _sc_body(f0_h, plab_h, mlab_h, ptab_h, mtab_h, w_h, bias_h, out_h,
             ptab_v, mtab_v, w_v, bias_v, labp_v, labm_v, f0_v,
             out0_v, out1_v, sem0, sem1):
    nc = 2
    wid = lax.axis_index("s") * nc + lax.axis_index("c")

    pltpu.sync_copy(ptab_h, ptab_v)
    pltpu.sync_copy(mtab_h, mtab_v)
    pltpu.sync_copy(w_h, w_v)
    pltpu.sync_copy(bias_h, bias_v)

    def compute(buf, off):
        fvs = [f0_v[pl.ds(off + sg * 16, 16)] for sg in range(SG)]

        @plsc.parallel_loop(0, F0_DIM // 16, unroll=4)
        def _(g):
            wvec = w_v[pl.ds(g * 16, 16)]
            bvec = bias_v[pl.ds(g * 16, 16)]
            for j in range(16):
                w = wvec[j]
                bb = bvec[j]
                for sg in range(SG):
                    buf[g * 16 + j, pl.ds(sg * 16, 16)] = fvs[sg] * w + bb

        basep = tuple(
            labp_v[pl.ds(off + sg * 16, 16)] * PH_STRIDE for sg in range(SG))

        @plsc.parallel_loop(0, PH_DIM, unroll=12, carry=basep)
        def _(d, idxs):
            for sg in range(SG):
                buf[F0_DIM + d, pl.ds(sg * 16, 16)] = plsc.load_gather(
                    ptab_v, [idxs[sg]])
            return tuple(ix + 1 for ix in idxs)

        basem = tuple(
            labm_v[pl.ds(off + sg * 16, 16)] * MIDI_STRIDE for sg in range(SG))

        @plsc.parallel_loop(0, MIDI_DIM, unroll=8, carry=basem)
        def _(d, idxs):
            for sg in range(SG):
                buf[F0_DIM + PH_DIM + d, pl.ds(sg * 16, 16)] = \
                    plsc.load_gather(mtab_v, [idxs[sg]])
            return tuple(ix + 1 for ix in idxs)

    n_iter = 2 * (N_CHUNK // 2)

    def body(i, _):
        b = wid * 2 + i // (N_CHUNK // 2)
        ic = i % (N_CHUNK // 2)

        @pl.when(ic == 0)
        def _():
            pltpu.sync_copy(plab_h.at[b], labp_v)
            pltpu.sync_copy(mlab_h.at[b], labm_v)
            pltpu.sync_copy(f0_h.at[b], f0_v)

        off0 = ic * (2 * CHUNK)

        @pl.when(i > 0)
        def _():
            pltpu.make_async_copy(
                out0_v, out_h.at[b, :, pl.ds(off0, CHUNK)], sem0).wait()

        compute(out0_v, off0)
        pltpu.async_copy(out0_v, out_h.at[b, :, pl.ds(off0, CHUNK)], sem0)

        @pl.when(i > 0)
        def _():
            pltpu.make_async_copy(
                out1_v, out_h.at[b, :, pl.ds(off0 + CHUNK, CHUNK)],
                sem1).wait()

        compute(out1_v, off0 + CHUNK)
        pltpu.async_copy(
            out1_v, out_h.at[b, :, pl.ds(off0 + CHUNK, CHUNK)], sem1)
        return 0

    lax.fori_loop(0, n_iter, body, 0)

    pltpu.make_async_copy(
        out0_v, out_h.at[0, :, pl.ds(0, CHUNK)], sem0).wait()
    pltpu.make_async_copy(
        out1_v, out_h.at[0, :, pl.ds(0, CHUNK)], sem1).wait()


@jax.jit
def _run(f0_2d, plab, mlab, ptab_flat, mtab_flat, w_flat, bias):
    mesh = plsc.VectorSubcoreMesh(core_axis_name="c", subcore_axis_name="s")
    sck = functools.partial(
        pl.kernel,
        mesh=mesh,
        compiler_params=pltpu.CompilerParams(needs_layout_passes=False),
        out_type=jax.ShapeDtypeStruct((B, D_TOT, S), jnp.float32),
        scratch_types=[
            pltpu.VMEM((NUM_PHONES * PH_STRIDE,), jnp.float32),
            pltpu.VMEM((NUM_MIDI * MIDI_STRIDE,), jnp.float32),
            pltpu.VMEM((F0_DIM,), jnp.float32),
            pltpu.VMEM((F0_DIM,), jnp.float32),
            pltpu.VMEM((S,), jnp.int32),
            pltpu.VMEM((S,), jnp.int32),
            pltpu.VMEM((S,), jnp.float32),
            pltpu.VMEM((D_TOT, CHUNK), jnp.float32),
            pltpu.VMEM((D_TOT, CHUNK), jnp.float32),
            pltpu.SemaphoreType.DMA,
            pltpu.SemaphoreType.DMA,
        ],
    )(_sc_body)
    return sck(f0_2d, plab, mlab, ptab_flat, mtab_flat, w_flat, bias)


def kernel(f0, phone_label, phone_duration, midi_label, W_f0, b_f0,
           phone_table, midi_table):
    del phone_duration
    f0_2d = f0.reshape(B, S)
    plab = phone_label.astype(jnp.int32)
    mlab = midi_label.astype(jnp.int32)
    ptab_pad = jnp.pad(phone_table, ((0, 0), (0, 1))).reshape(-1)
    mtab_pad = jnp.pad(midi_table, ((0, 0), (0, 1))).reshape(-1)
    return _run(f0_2d, plab, mlab, ptab_pad, mtab_pad,
                W_f0.reshape(-1), b_f0)


# speedup vs baseline: 1.3650x; 1.1915x over previous
import functools

import jax
import jax.numpy as jnp
from jax import lax
from jax.experimental import pallas as pl
from jax.experimental.pallas import tpu as pltpu
from jax.experimental.pallas import tpu_sc as plsc

B, S = 64, 2048
F0_DIM, PH_DIM, MIDI_DIM = 64, 192, 64
NUM_PHONES, NUM_MIDI = 100, 128
D_TOT = F0_DIM + PH_DIM + MIDI_DIM
PH_STRIDE, MIDI_STRIDE = PH_DIM + 1, MIDI_DIM + 1

CHUNK = 128
SG = CHUNK // 16
N_CHUNK = S // CHUNK


def _sc_body(f0_h, plab_h, mlab_h, ptab_h, mtab_h, w_h, bias_h, out_h,
             ptab_v, mtab_v, w_v, bias_v, labp_v, labm_v, f0_v,
             out0_v, out1_v, sem0, sem1):
    nc = 2
    wid = lax.axis_index("s") * nc + lax.axis_index("c")

    pltpu.sync_copy(ptab_h, ptab_v)
    pltpu.sync_copy(mtab_h, mtab_v)
    pltpu.sync_copy(w_h, w_v)
    pltpu.sync_copy(bias_h, bias_v)

    def compute(buf, off):
        fvs = [f0_v[pl.ds(off + sg * 16, 16)] for sg in range(SG)]

        @plsc.parallel_loop(0, F0_DIM // 16, unroll=2)
        def _(g):
            wvec = w_v[pl.ds(g * 16, 16)]
            bvec = bias_v[pl.ds(g * 16, 16)]
            for j in range(16):
                w = wvec[j]
                bb = bvec[j]
                for sg in range(SG):
                    buf[g * 16 + j, pl.ds(sg * 16, 16)] = fvs[sg] * w + bb

        basep = tuple(
            labp_v[pl.ds(off + sg * 16, 16)] * PH_STRIDE for sg in range(SG))

        @plsc.parallel_loop(0, PH_DIM, unroll=10, carry=basep)
        def _(d, idxs):
            for sg in range(SG):
                buf[F0_DIM + d, pl.ds(sg * 16, 16)] = plsc.load_gather(
                    ptab_v, [idxs[sg]])
            return tuple(ix + 1 for ix in idxs)

        basem = tuple(
            labm_v[pl.ds(off + sg * 16, 16)] * MIDI_STRIDE for sg in range(SG))

        @plsc.parallel_loop(0, MIDI_DIM, unroll=8, carry=basem)
        def _(d, idxs):
            for sg in range(SG):
                buf[F0_DIM + PH_DIM + d, pl.ds(sg * 16, 16)] = \
                    plsc.load_gather(mtab_v, [idxs[sg]])
            return tuple(ix + 1 for ix in idxs)

    n_iter = 2 * (N_CHUNK // 2)

    def body(i, _):
        b = wid * 2 + i // (N_CHUNK // 2)
        ic = i % (N_CHUNK // 2)

        @pl.when(ic == 0)
        def _():
            pltpu.sync_copy(plab_h.at[b], labp_v)
            pltpu.sync_copy(mlab_h.at[b], labm_v)
            pltpu.sync_copy(f0_h.at[b], f0_v)

        off0 = ic * (2 * CHUNK)

        @pl.when(i > 0)
        def _():
            pltpu.make_async_copy(
                out0_v, out_h.at[b, :, pl.ds(off0, CHUNK)], sem0).wait()

        compute(out0_v, off0)
        pltpu.async_copy(out0_v, out_h.at[b, :, pl.ds(off0, CHUNK)], sem0)

        @pl.when(i > 0)
        def _():
            pltpu.make_async_copy(
                out1_v, out_h.at[b, :, pl.ds(off0 + CHUNK, CHUNK)],
                sem1).wait()

        compute(out1_v, off0 + CHUNK)
        pltpu.async_copy(
            out1_v, out_h.at[b, :, pl.ds(off0 + CHUNK, CHUNK)], sem1)
        return 0

    lax.fori_loop(0, n_iter, body, 0)

    pltpu.make_async_copy(
        out0_v, out_h.at[0, :, pl.ds(0, CHUNK)], sem0).wait()
    pltpu.make_async_copy(
        out1_v, out_h.at[0, :, pl.ds(0, CHUNK)], sem1).wait()


@jax.jit
def _run(f0_2d, plab, mlab, ptab_flat, mtab_flat, w_flat, bias):
    mesh = plsc.VectorSubcoreMesh(core_axis_name="c", subcore_axis_name="s")
    sck = functools.partial(
        pl.kernel,
        mesh=mesh,
        compiler_params=pltpu.CompilerParams(needs_layout_passes=False),
        out_type=jax.ShapeDtypeStruct((B, D_TOT, S), jnp.float32),
        scratch_types=[
            pltpu.VMEM((NUM_PHONES * PH_STRIDE,), jnp.float32),
            pltpu.VMEM((NUM_MIDI * MIDI_STRIDE,), jnp.float32),
            pltpu.VMEM((F0_DIM,), jnp.float32),
            pltpu.VMEM((F0_DIM,), jnp.float32),
            pltpu.VMEM((S,), jnp.int32),
            pltpu.VMEM((S,), jnp.int32),
            pltpu.VMEM((S,), jnp.float32),
            pltpu.VMEM((D_TOT, CHUNK), jnp.float32),
            pltpu.VMEM((D_TOT, CHUNK), jnp.float32),
            pltpu.SemaphoreType.DMA,
            pltpu.SemaphoreType.DMA,
        ],
    )(_sc_body)
    return sck(f0_2d, plab, mlab, ptab_flat, mtab_flat, w_flat, bias)


def kernel(f0, phone_label, phone_duration, midi_label, W_f0, b_f0,
           phone_table, midi_table):
    del phone_duration
    f0_2d = f0.reshape(B, S)
    plab = phone_label.astype(jnp.int32)
    mlab = midi_label.astype(jnp.int32)
    ptab_pad = jnp.pad(phone_table, ((0, 0), (0, 1))).reshape(-1)
    mtab_pad = jnp.pad(midi_table, ((0, 0), (0, 1))).reshape(-1)
    return _run(f0_2d, plab, mlab, ptab_pad, mtab_pad,
                W_f0.reshape(-1), b_f0)


# speedup vs baseline: 1.3980x; 1.0242x over previous
import functools

import jax
import jax.numpy as jnp
from jax import lax
from jax.experimental import pallas as pl
from jax.experimental.pallas import tpu as pltpu
from jax.experimental.pallas import tpu_sc as plsc

B, S = 64, 2048
F0_DIM, PH_DIM, MIDI_DIM = 64, 192, 64
NUM_PHONES, NUM_MIDI = 100, 128
D_TOT = F0_DIM + PH_DIM + MIDI_DIM
PH_STRIDE, MIDI_STRIDE = PH_DIM + 1, MIDI_DIM + 1

CHUNK = 128
SG = CHUNK // 16
N_CHUNK = S // CHUNK


def _sc_body(f0_h, plab_h, mlab_h, ptab_h, mtab_h, w_h, bias_h, out_h,
             ptab_v, mtab_v, w_v, bias_v, labp_v, labm_v, f0_v,
             out0_v, out1_v, sem0, sem1):
    nc = 2
    wid = lax.axis_index("s") * nc + lax.axis_index("c")

    pltpu.sync_copy(ptab_h, ptab_v)
    pltpu.sync_copy(mtab_h, mtab_v)
    pltpu.sync_copy(w_h, w_v)
    pltpu.sync_copy(bias_h, bias_v)

    def compute(buf, off):
        fvs = [f0_v[pl.ds(off + sg * 16, 16)] for sg in range(SG)]

        @plsc.parallel_loop(0, F0_DIM // 16, unroll=2)
        def _(g):
            wvec = w_v[pl.ds(g * 16, 16)]
            bvec = bias_v[pl.ds(g * 16, 16)]
            for j in range(16):
                w = wvec[j]
                bb = bvec[j]
                for sg in range(SG):
                    buf[g * 16 + j, pl.ds(sg * 16, 16)] = fvs[sg] * w + bb

        basep = tuple(
            labp_v[pl.ds(off + sg * 16, 16)] * PH_STRIDE for sg in range(SG))

        @plsc.parallel_loop(0, PH_DIM, unroll=12, carry=basep)
        def _(d, idxs):
            for sg in range(SG):
                buf[F0_DIM + d, pl.ds(sg * 16, 16)] = plsc.load_gather(
                    ptab_v, [idxs[sg]])
            return tuple(ix + 1 for ix in idxs)

        basem = tuple(
            labm_v[pl.ds(off + sg * 16, 16)] * MIDI_STRIDE for sg in range(SG))

        @plsc.parallel_loop(0, MIDI_DIM, unroll=8, carry=basem)
        def _(d, idxs):
            for sg in range(SG):
                buf[F0_DIM + PH_DIM + d, pl.ds(sg * 16, 16)] = \
                    plsc.load_gather(mtab_v, [idxs[sg]])
            return tuple(ix + 1 for ix in idxs)

    n_iter = 2 * (N_CHUNK // 2)

    def body(i, _):
        b = wid * 2 + i // (N_CHUNK // 2)
        ic = i % (N_CHUNK // 2)

        @pl.when(ic == 0)
        def _():
            pltpu.sync_copy(plab_h.at[b], labp_v)
            pltpu.sync_copy(mlab_h.at[b], labm_v)
            pltpu.sync_copy(f0_h.at[b], f0_v)

        off0 = ic * (2 * CHUNK)

        @pl.when(i > 0)
        def _():
            pltpu.make_async_copy(
                out0_v, out_h.at[b, :, pl.ds(off0, CHUNK)], sem0).wait()

        compute(out0_v, off0)
        pltpu.async_copy(out0_v, out_h.at[b, :, pl.ds(off0, CHUNK)], sem0)

        @pl.when(i > 0)
        def _():
            pltpu.make_async_copy(
                out1_v, out_h.at[b, :, pl.ds(off0 + CHUNK, CHUNK)],
                sem1).wait()

        compute(out1_v, off0 + CHUNK)
        pltpu.async_copy(
            out1_v, out_h.at[b, :, pl.ds(off0 + CHUNK, CHUNK)], sem1)
        return 0

    lax.fori_loop(0, n_iter, body, 0)

    pltpu.make_async_copy(
        out0_v, out_h.at[0, :, pl.ds(0, CHUNK)], sem0).wait()
    pltpu.make_async_copy(
        out1_v, out_h.at[0, :, pl.ds(0, CHUNK)], sem1).wait()


@jax.jit
def _run(f0_2d, plab, mlab, ptab_flat, mtab_flat, w_flat, bias):
    mesh = plsc.VectorSubcoreMesh(core_axis_name="c", subcore_axis_name="s")
    sck = functools.partial(
        pl.kernel,
        mesh=mesh,
        compiler_params=pltpu.CompilerParams(needs_layout_passes=False),
        out_type=jax.ShapeDtypeStruct((B, D_TOT, S), jnp.float32),
        scratch_types=[
            pltpu.VMEM((NUM_PHONES * PH_STRIDE,), jnp.float32),
            pltpu.VMEM((NUM_MIDI * MIDI_STRIDE,), jnp.float32),
            pltpu.VMEM((F0_DIM,), jnp.float32),
            pltpu.VMEM((F0_DIM,), jnp.float32),
            pltpu.VMEM((S,), jnp.int32),
            pltpu.VMEM((S,), jnp.int32),
            pltpu.VMEM((S,), jnp.float32),
            pltpu.VMEM((D_TOT, CHUNK), jnp.float32),
            pltpu.VMEM((D_TOT, CHUNK), jnp.float32),
            pltpu.SemaphoreType.DMA,
            pltpu.SemaphoreType.DMA,
        ],
    )(_sc_body)
    return sck(f0_2d, plab, mlab, ptab_flat, mtab_flat, w_flat, bias)


def kernel(f0, phone_label, phone_duration, midi_label, W_f0, b_f0,
           phone_table, midi_table):
    del phone_duration
    f0_2d = f0.reshape(B, S)
    plab = phone_label.astype(jnp.int32)
    mlab = midi_label.astype(jnp.int32)
    ptab_pad = jnp.pad(phone_table, ((0, 0), (0, 1))).reshape(-1)
    mtab_pad = jnp.pad(midi_table, ((0, 0), (0, 1))).reshape(-1)
    return _run(f0_2d, plab, mlab, ptab_pad, mtab_pad,
                W_f0.reshape(-1), b_f0)
